# baseline (speedup 1.0000x reference)
import jax
import jax.numpy as jnp
from jax import lax
from jax.experimental import pallas as pl
from jax.experimental.pallas import tpu as pltpu
from jax.experimental.pallas import tpu_sc as plsc

VOCAB = 1000000
HIDDEN = 32
EMB = 128
BATCH = 16384
HIST = 50
NTOK = BATCH * HIST


_WBLK = 16384


def _wb_body(ut_ref, v_ref, b_ref, w_ref):
    w_ref[...] = (
        lax.dot_general(
            ut_ref[...], v_ref[...],
            (((0,), (0,)), ((), ())),
            preferred_element_type=jnp.float32,
        )
        + b_ref[...]
    )


def _build_w(ut, v, b):
    return pl.pallas_call(
        _wb_body,
        grid=((VOCAB + _WBLK - 1) // _WBLK,),
        in_specs=[
            pl.BlockSpec((HIDDEN, _WBLK), lambda i: (0, i)),
            pl.BlockSpec((HIDDEN, EMB), lambda i: (0, 0)),
            pl.BlockSpec((1, EMB), lambda i: (0, 0)),
        ],
        out_specs=pl.BlockSpec((_WBLK, EMB), lambda i: (i, 0)),
        out_shape=jax.ShapeDtypeStruct((VOCAB, EMB), jnp.float32),
    )(ut, v, b.reshape(1, EMB))


_INFO = plsc.get_sparse_core_info()
_NC = _INFO.num_cores
_NS = _INFO.num_subcores
_NW = _NC * _NS
_TOK_PER_W = NTOK // _NW
_CHUNK = 400
_NCHUNK = _TOK_PER_W // _CHUNK
_GATHERS = ((0, 128), (128, 128), (256, 128), (384, 16))


def _sc_body(idx_hbm, w_hbm, out_hbm,
             idx_all, rows0, rows1, sg0, sg1, sw0, sw1):
    wid = lax.axis_index("s") * _NC + lax.axis_index("c")
    base = wid * _TOK_PER_W

    pltpu.sync_copy(idx_hbm.at[pl.ds(base, _TOK_PER_W)], idx_all)

    def g_descs(c, rowsv, sem):
        return [
            pltpu.make_async_copy(
                w_hbm.at[idx_all.at[pl.ds(c * _CHUNK + o, n)]],
                rowsv.at[pl.ds(o, n)],
                sem,
            )
            for o, n in _GATHERS
        ]

    def w_desc(c, rowsv, sem):
        return pltpu.make_async_copy(
            rowsv, out_hbm.at[pl.ds(base + c * _CHUNK, _CHUNK)], sem)

    def pair(p, carry):
        c0 = 2 * p

        @pl.when(p >= 1)
        def _():
            w_desc(c0 - 2, rows0, sw0).wait()

        for d in g_descs(c0, rows0, sg0):
            d.start()

        @pl.when(p >= 1)
        def _():
            for d in g_descs(c0 - 1, rows1, sg1):
                d.wait()
            w_desc(c0 - 1, rows1, sw1).start()
            w_desc(c0 - 1, rows1, sw1).wait()

        for d in g_descs(c0 + 1, rows1, sg1):
            d.start()

        for d in g_descs(c0, rows0, sg0):
            d.wait()
        w_desc(c0, rows0, sw0).start()
        return carry

    lax.fori_loop(0, _NCHUNK // 2, pair, 0)

    for d in g_descs(_NCHUNK - 1, rows1, sg1):
        d.wait()
    w_desc(_NCHUNK - 1, rows1, sw1).start()
    w_desc(_NCHUNK - 2, rows0, sw0).wait()
    w_desc(_NCHUNK - 1, rows1, sw1).wait()


def _sc_gather(idx_flat, w):
    mesh = plsc.VectorSubcoreMesh(core_axis_name="c", subcore_axis_name="s")
    k = pl.kernel(
        _sc_body,
        out_type=jax.ShapeDtypeStruct((NTOK, EMB), jnp.float32),
        mesh=mesh,
        scratch_types=[
            pltpu.VMEM((_TOK_PER_W,), jnp.int32),
            pltpu.VMEM((_CHUNK, EMB), jnp.float32),
            pltpu.VMEM((_CHUNK, EMB), jnp.float32),
            pltpu.SemaphoreType.DMA,
            pltpu.SemaphoreType.DMA,
            pltpu.SemaphoreType.DMA,
            pltpu.SemaphoreType.DMA,
        ],
        compiler_params=pltpu.CompilerParams(use_tc_tiling_on_sc=True),
    )
    return k(idx_flat, w)


@jax.jit
def kernel(x, u_weight, v_weight, v_bias):
    idx_flat = x.T.reshape(-1).astype(jnp.int32)
    w = _build_w(u_weight.T, v_weight, v_bias)
    out2d = _sc_gather(idx_flat, w)
    return out2d.reshape(HIST, BATCH, EMB).transpose(1, 0, 2)


# speedup vs baseline: 1.0089x; 1.0089x over previous
import jax
import jax.numpy as jnp
from jax import lax
from jax.experimental import pallas as pl
from jax.experimental.pallas import tpu as pltpu
from jax.experimental.pallas import tpu_sc as plsc

VOCAB = 1000000
HIDDEN = 32
EMB = 128
BATCH = 16384
HIST = 50
NTOK = BATCH * HIST


_WBLK = 32768


def _wb_body(ut_ref, v_ref, b_ref, w_ref):
    w_ref[...] = (
        lax.dot_general(
            ut_ref[...], v_ref[...],
            (((0,), (0,)), ((), ())),
            preferred_element_type=jnp.float32,
        )
        + b_ref[...]
    )


def _build_w(ut, v, b):
    return pl.pallas_call(
        _wb_body,
        grid=((VOCAB + _WBLK - 1) // _WBLK,),
        in_specs=[
            pl.BlockSpec((HIDDEN, _WBLK), lambda i: (0, i)),
            pl.BlockSpec((HIDDEN, EMB), lambda i: (0, 0)),
            pl.BlockSpec((1, EMB), lambda i: (0, 0)),
        ],
        out_specs=pl.BlockSpec((_WBLK, EMB), lambda i: (i, 0)),
        out_shape=jax.ShapeDtypeStruct((VOCAB, EMB), jnp.float32),
    )(ut, v, b.reshape(1, EMB))


_INFO = plsc.get_sparse_core_info()
_NC = _INFO.num_cores
_NS = _INFO.num_subcores
_NW = _NC * _NS
_TOK_PER_W = NTOK // _NW
_CHUNK = 400
_NCHUNK = _TOK_PER_W // _CHUNK
_GATHERS = ((0, 128), (128, 128), (256, 128), (384, 16))


def _sc_body(idx_hbm, w_hbm, out_hbm,
             idx_all, rows0, rows1, sg0, sg1, sw0, sw1):
    wid = lax.axis_index("s") * _NC + lax.axis_index("c")
    base = wid * _TOK_PER_W

    pltpu.sync_copy(idx_hbm.at[pl.ds(base, _TOK_PER_W)], idx_all)

    def g_descs(c, rowsv, sem):
        return [
            pltpu.make_async_copy(
                w_hbm.at[idx_all.at[pl.ds(c * _CHUNK + o, n)]],
                rowsv.at[pl.ds(o, n)],
                sem,
            )
            for o, n in _GATHERS
        ]

    def w_desc(c, rowsv, sem):
        return pltpu.make_async_copy(
            rowsv, out_hbm.at[pl.ds(base + c * _CHUNK, _CHUNK)], sem)

    def pair(p, carry):
        c0 = 2 * p

        @pl.when(p >= 1)
        def _():
            w_desc(c0 - 2, rows0, sw0).wait()

        for d in g_descs(c0, rows0, sg0):
            d.start()

        @pl.when(p >= 1)
        def _():
            for d in g_descs(c0 - 1, rows1, sg1):
                d.wait()
            w_desc(c0 - 1, rows1, sw1).start()
            w_desc(c0 - 1, rows1, sw1).wait()

        for d in g_descs(c0 + 1, rows1, sg1):
            d.start()

        for d in g_descs(c0, rows0, sg0):
            d.wait()
        w_desc(c0, rows0, sw0).start()
        return carry

    lax.fori_loop(0, _NCHUNK // 2, pair, 0)

    for d in g_descs(_NCHUNK - 1, rows1, sg1):
        d.wait()
    w_desc(_NCHUNK - 1, rows1, sw1).start()
    w_desc(_NCHUNK - 2, rows0, sw0).wait()
    w_desc(_NCHUNK - 1, rows1, sw1).wait()


def _sc_gather(idx_flat, w):
    mesh = plsc.VectorSubcoreMesh(core_axis_name="c", subcore_axis_name="s")
    k = pl.kernel(
        _sc_body,
        out_type=jax.ShapeDtypeStruct((NTOK, EMB), jnp.float32),
        mesh=mesh,
        scratch_types=[
            pltpu.VMEM((_TOK_PER_W,), jnp.int32),
            pltpu.VMEM((_CHUNK, EMB), jnp.float32),
            pltpu.VMEM((_CHUNK, EMB), jnp.float32),
            pltpu.SemaphoreType.DMA,
            pltpu.SemaphoreType.DMA,
            pltpu.SemaphoreType.DMA,
            pltpu.SemaphoreType.DMA,
        ],
        compiler_params=pltpu.CompilerParams(use_tc_tiling_on_sc=True),
    )
    return k(idx_flat, w)


@jax.jit
def kernel(x, u_weight, v_weight, v_bias):
    idx_flat = x.T.reshape(-1).astype(jnp.int32)
    w = _build_w(u_weight.T, v_weight, v_bias)
    out2d = _sc_gather(idx_flat, w)
    return out2d.reshape(HIST, BATCH, EMB).transpose(1, 0, 2)


# speedup vs baseline: 1.0099x; 1.0010x over previous
import jax
import jax.numpy as jnp
from jax import lax
from jax.experimental import pallas as pl
from jax.experimental.pallas import tpu as pltpu
from jax.experimental.pallas import tpu_sc as plsc

VOCAB = 1000000
HIDDEN = 32
EMB = 128
BATCH = 16384
HIST = 50
NTOK = BATCH * HIST


_WBLK = 32768


def _wb_body(ut_ref, v_ref, b_ref, w_ref):
    w_ref[...] = (
        lax.dot_general(
            ut_ref[...], v_ref[...],
            (((0,), (0,)), ((), ())),
            preferred_element_type=jnp.float32,
        )
        + b_ref[...]
    )


def _build_w(ut, v, b):
    return pl.pallas_call(
        _wb_body,
        grid=((VOCAB + _WBLK - 1) // _WBLK,),
        in_specs=[
            pl.BlockSpec((HIDDEN, _WBLK), lambda i: (0, i)),
            pl.BlockSpec((HIDDEN, EMB), lambda i: (0, 0)),
            pl.BlockSpec((1, EMB), lambda i: (0, 0)),
        ],
        out_specs=pl.BlockSpec((_WBLK, EMB), lambda i: (i, 0)),
        out_shape=jax.ShapeDtypeStruct((VOCAB, EMB), jnp.float32),
    )(ut, v, b.reshape(1, EMB))


_INFO = plsc.get_sparse_core_info()
_NC = _INFO.num_cores
_NS = _INFO.num_subcores
_NW = _NC * _NS
_TOK_PER_W = NTOK // _NW
_CHUNK = 256
_NCHUNK = _TOK_PER_W // _CHUNK
_GATHERS = ((0, 128), (128, 128))
_NTRIPLE = (_NCHUNK - 1) // 3


def _sc_body(idx_hbm, w_hbm, out_hbm,
             idx_all, rows0, rows1, rows2, sg0, sg1, sg2, sw0, sw1, sw2):
    wid = lax.axis_index("s") * _NC + lax.axis_index("c")
    base = wid * _TOK_PER_W
    rows = (rows0, rows1, rows2)
    sg = (sg0, sg1, sg2)
    sw = (sw0, sw1, sw2)

    pltpu.sync_copy(idx_hbm.at[pl.ds(base, _TOK_PER_W)], idx_all)

    def g_descs(c, j):
        return [
            pltpu.make_async_copy(
                w_hbm.at[idx_all.at[pl.ds(c * _CHUNK + o, n)]],
                rows[j].at[pl.ds(o, n)],
                sg[j],
            )
            for o, n in _GATHERS
        ]

    def w_desc(c, j):
        return pltpu.make_async_copy(
            rows[j], out_hbm.at[pl.ds(base + c * _CHUNK, _CHUNK)], sw[j])

    def triple(p, carry):
        c0 = 3 * p
        for k in range(3):
            c = c0 + k

            @pl.when(c >= 3)
            def _():
                w_desc(c - 3, k).wait()

            for d in g_descs(c, k):
                d.start()

            @pl.when(c >= 1)
            def _():
                for d in g_descs(c - 1, (k + 2) % 3):
                    d.wait()
                w_desc(c - 1, (k + 2) % 3).start()

        return carry

    lax.fori_loop(0, _NTRIPLE, triple, 0)

    last = 3 * _NTRIPLE
    for d in g_descs(last - 1, 2):
        d.wait()
    w_desc(last - 1, 2).start()
    w_desc(last - 3, 0).wait()
    for d in g_descs(last, 0):
        d.start()
    for d in g_descs(last, 0):
        d.wait()
    w_desc(last, 0).start()
    w_desc(last - 2, 1).wait()
    w_desc(last - 1, 2).wait()
    w_desc(last, 0).wait()


def _sc_gather(idx_flat, w):
    mesh = plsc.VectorSubcoreMesh(core_axis_name="c", subcore_axis_name="s")
    k = pl.kernel(
        _sc_body,
        out_type=jax.ShapeDtypeStruct((NTOK, EMB), jnp.float32),
        mesh=mesh,
        scratch_types=[
            pltpu.VMEM((_TOK_PER_W,), jnp.int32),
            pltpu.VMEM((_CHUNK, EMB), jnp.float32),
            pltpu.VMEM((_CHUNK, EMB), jnp.float32),
            pltpu.VMEM((_CHUNK, EMB), jnp.float32),
            pltpu.SemaphoreType.DMA,
            pltpu.SemaphoreType.DMA,
            pltpu.SemaphoreType.DMA,
            pltpu.SemaphoreType.DMA,
            pltpu.SemaphoreType.DMA,
            pltpu.SemaphoreType.DMA,
        ],
        compiler_params=pltpu.CompilerParams(use_tc_tiling_on_sc=True),
    )
    return k(idx_flat, w)


@jax.jit
def kernel(x, u_weight, v_weight, v_bias):
    idx_flat = x.T.reshape(-1).astype(jnp.int32)
    w = _build_w(u_weight.T, v_weight, v_bias)
    out2d = _sc_gather(idx_flat, w)
    return out2d.reshape(HIST, BATCH, EMB).transpose(1, 0, 2)
